# baseline (speedup 1.0000x reference)
import functools

import jax
import jax.numpy as jnp
from jax import lax
from jax.experimental import pallas as pl
from jax.experimental.pallas import tpu as pltpu
from jax.experimental.pallas import tpu_sc as plsc

N = 10000
E = 320000
D = 128
NC, NS = 2, 16
NW = NC * NS
EPW = E // NW
CH = 80
NCHUNK = EPW // CH
NP = 10240
STRIPE = NP // NS
NF = NP // D

_mesh = plsc.VectorSubcoreMesh(core_axis_name="c", subcore_axis_name="s")
_sc_params = pltpu.CompilerParams(use_tc_tiling_on_sc=False)


@functools.partial(
    pl.kernel,
    out_type=jax.ShapeDtypeStruct((NC, NF, D), jnp.float32),
    mesh=_mesh,
    compiler_params=_sc_params,
    scratch_types=[
        pltpu.VMEM_SHARED((NP,), jnp.float32),
        pltpu.VMEM((NCHUNK, CH), jnp.int32),
        pltpu.VMEM((CH,), jnp.float32),
        pltpu.VMEM((1024,), jnp.float32),
        pltpu.VMEM((8, D), jnp.float32),
        pltpu.SemaphoreType.DMA,
    ],
)
def _deg_kernel(row3, ones_hbm, zflat_hbm, degp, deg_sh, idx_v, ones_v,
                flat_v, pack_v, sem):
    c = lax.axis_index("c")
    s = lax.axis_index("s")
    wid = c * NS + s
    pltpu.sync_copy(zflat_hbm, deg_sh.at[pl.ds(s * STRIPE, STRIPE)])
    pltpu.sync_copy(row3.at[wid], idx_v)
    pltpu.sync_copy(ones_hbm, ones_v)
    plsc.subcore_barrier()

    def hist_body(j, carry):
        pltpu.async_copy(ones_v, deg_sh.at[idx_v.at[j]], sem, add=True)
        return carry

    lax.fori_loop(0, NCHUNK, hist_body, 0)

    def hist_drain(j, carry):
        pltpu.make_async_copy(ones_v, deg_sh.at[idx_v.at[0]], sem).wait()
        return carry

    lax.fori_loop(0, NCHUNK, hist_drain, 0)
    plsc.subcore_barrier()

    @pl.when(s < 10)
    def _():
        pltpu.sync_copy(deg_sh.at[pl.ds(s * 1024, 1024)], flat_v)
        for r in range(8):
            for k in range(D // 16):
                pack_v[r, pl.ds(k * 16, 16)] = flat_v[pl.ds(r * D + k * 16, 16)]
        pltpu.sync_copy(pack_v, degp.at[c].at[pl.ds(s * 8, 8)])


NBUF = 12


@functools.partial(
    pl.kernel,
    out_type=jax.ShapeDtypeStruct((NC, NP, D), jnp.bfloat16),
    mesh=_mesh,
    compiler_params=_sc_params,
    scratch_types=[
        pltpu.VMEM_SHARED((NP, D), jnp.bfloat16),
        pltpu.VMEM((NCHUNK, CH), jnp.int32),
        pltpu.VMEM((NCHUNK, CH), jnp.int32),
        pltpu.VMEM((NBUF, CH, D), jnp.bfloat16),
        pltpu.SemaphoreType.DMA,
        pltpu.SemaphoreType.DMA,
    ],
)
def _agg_kernel(y_hbm, row3, col3, zrows_hbm, aggp, agg_sh, row_v, col_v,
                bufs, gsem, ssem):
    c = lax.axis_index("c")
    s = lax.axis_index("s")
    wid = c * NS + s
    pltpu.sync_copy(zrows_hbm, agg_sh.at[pl.ds(s * STRIPE, STRIPE)])
    pltpu.sync_copy(row3.at[wid], row_v)
    pltpu.sync_copy(col3.at[wid], col_v)
    plsc.subcore_barrier()

    def gather_start(j, b):
        pltpu.async_copy(y_hbm.at[row_v.at[j]], bufs.at[b], gsem)

    def gather_wait(b):
        pltpu.make_async_copy(y_hbm.at[row_v.at[0]], bufs.at[b], gsem).wait()

    def scatter_start(j, b):
        pltpu.async_copy(bufs.at[b], agg_sh.at[col_v.at[j]], ssem, add=True)

    def scatter_wait():
        pltpu.make_async_copy(bufs.at[0], agg_sh.at[col_v.at[0]],
                              ssem).wait()

    for j in range(NBUF - 1):
        gather_start(j, j)

    def body(g, carry):
        for b in range(NBUF):
            j = NBUF * g + b
            gather_wait(b)
            scatter_start(j, b)
            scatter_wait()
            gather_start(j + NBUF - 1, (b + NBUF - 1) % NBUF)
        return carry

    NB_STEPS = (NCHUNK - NBUF) // NBUF
    lax.fori_loop(0, NB_STEPS, body, 0)
    done = NB_STEPS * NBUF
    for j in range(done, NCHUNK):
        b = j % NBUF
        gather_wait(b)
        scatter_start(j, b)
        scatter_wait()
        if j + NBUF - 1 < NCHUNK:
            gather_start(j + NBUF - 1, (b + NBUF - 1) % NBUF)
    plsc.subcore_barrier()
    pltpu.sync_copy(agg_sh.at[pl.ds(s * STRIPE, STRIPE)],
                    aggp.at[c].at[pl.ds(s * STRIPE, STRIPE)])


_RB = 1024
_RF = _RB // D


def _discol(degp_ref):
    deg = degp_ref[0] + degp_ref[1] + 1.0
    dis = lax.rsqrt(deg)
    t = jnp.repeat(dis, D, axis=0)
    rr = lax.broadcasted_iota(jnp.int32, (_RB, D), 0)
    cc = lax.broadcasted_iota(jnp.int32, (_RB, D), 1)
    eye = jnp.where(rr % D == cc, 1.0, 0.0)
    return jnp.sum(t * eye, axis=1, keepdims=True)


def _scale_body(x_ref, degp_ref, y_ref):
    y_ref[...] = (x_ref[...] * _discol(degp_ref)).astype(jnp.bfloat16)


def _out_body(aggp_ref, degp_ref, x_ref, w_ref, b_ref, out_ref):
    dis = _discol(degp_ref)
    aggsum = (aggp_ref[0].astype(jnp.float32) +
              aggp_ref[1].astype(jnp.float32))
    m = dis * aggsum + (dis * dis) * x_ref[...]
    out_ref[...] = lax.dot_general(
        m, w_ref[...], (((1,), (0,)), ((), ())),
        preferred_element_type=jnp.float32) + b_ref[...]


_GRID = NP // _RB


def kernel(x, edge_index, W, b):
    row3 = edge_index[0].reshape(NW, NCHUNK, CH)
    col3 = edge_index[1].reshape(NW, NCHUNK, CH)
    ones_f = jnp.ones((CH,), jnp.float32)
    zflat = jnp.zeros((STRIPE,), jnp.float32)
    zrows = jnp.zeros((STRIPE, D), jnp.bfloat16)

    degp = _deg_kernel(row3, ones_f, zflat)

    y = pl.pallas_call(
        _scale_body,
        grid=(_GRID,),
        in_specs=[
            pl.BlockSpec((_RB, D), lambda i: (i, 0)),
            pl.BlockSpec((NC, _RF, D), lambda i: (0, i, 0)),
        ],
        out_specs=pl.BlockSpec((_RB, D), lambda i: (i, 0)),
        out_shape=jax.ShapeDtypeStruct((N, D), jnp.bfloat16),
    )(x, degp)

    aggp = _agg_kernel(y, row3, col3, zrows)

    out = pl.pallas_call(
        _out_body,
        grid=(_GRID,),
        in_specs=[
            pl.BlockSpec((NC, _RB, D), lambda i: (0, i, 0)),
            pl.BlockSpec((NC, _RF, D), lambda i: (0, i, 0)),
            pl.BlockSpec((_RB, D), lambda i: (i, 0)),
            pl.BlockSpec((D, D), lambda i: (0, 0)),
            pl.BlockSpec((1, D), lambda i: (0, 0)),
        ],
        out_specs=pl.BlockSpec((_RB, D), lambda i: (i, 0)),
        out_shape=jax.ShapeDtypeStruct((N, D), jnp.float32),
    )(aggp, degp, x, W, b.reshape(1, D))
    return out


# speedup vs baseline: 1.0056x; 1.0056x over previous
import functools

import jax
import jax.numpy as jnp
from jax import lax
from jax.experimental import pallas as pl
from jax.experimental.pallas import tpu as pltpu
from jax.experimental.pallas import tpu_sc as plsc

N = 10000
E = 320000
D = 128
NC, NS = 2, 16
NW = NC * NS
EPW = E // NW
CH = 80
NCHUNK = EPW // CH
NP = 10240
STRIPE = NP // NS
NF = NP // D

_mesh = plsc.VectorSubcoreMesh(core_axis_name="c", subcore_axis_name="s")
_sc_params = pltpu.CompilerParams(use_tc_tiling_on_sc=False)


@functools.partial(
    pl.kernel,
    out_type=jax.ShapeDtypeStruct((NC, NF, D), jnp.float32),
    mesh=_mesh,
    compiler_params=_sc_params,
    scratch_types=[
        pltpu.VMEM_SHARED((NP,), jnp.float32),
        pltpu.VMEM((NCHUNK, CH), jnp.int32),
        pltpu.VMEM((CH,), jnp.float32),
        pltpu.VMEM((1024,), jnp.float32),
        pltpu.VMEM((8, D), jnp.float32),
        pltpu.SemaphoreType.DMA,
    ],
)
def _deg_kernel(row3, ones_hbm, zflat_hbm, degp, deg_sh, idx_v, ones_v,
                flat_v, pack_v, sem):
    c = lax.axis_index("c")
    s = lax.axis_index("s")
    wid = c * NS + s
    pltpu.sync_copy(zflat_hbm, deg_sh.at[pl.ds(s * STRIPE, STRIPE)])
    pltpu.sync_copy(row3.at[wid], idx_v)
    pltpu.sync_copy(ones_hbm, ones_v)
    plsc.subcore_barrier()

    def hist_body(j, carry):
        pltpu.async_copy(ones_v, deg_sh.at[idx_v.at[j]], sem, add=True)
        return carry

    lax.fori_loop(0, NCHUNK, hist_body, 0)

    def hist_drain(j, carry):
        pltpu.make_async_copy(ones_v, deg_sh.at[idx_v.at[0]], sem).wait()
        return carry

    lax.fori_loop(0, NCHUNK, hist_drain, 0)
    plsc.subcore_barrier()

    @pl.when(s < 10)
    def _():
        pltpu.sync_copy(deg_sh.at[pl.ds(s * 1024, 1024)], flat_v)
        for r in range(8):
            for k in range(D // 16):
                pack_v[r, pl.ds(k * 16, 16)] = flat_v[pl.ds(r * D + k * 16, 16)]
        pltpu.sync_copy(pack_v, degp.at[c].at[pl.ds(s * 8, 8)])


NBUF = 7


@functools.partial(
    pl.kernel,
    out_type=jax.ShapeDtypeStruct((NC, NP, D), jnp.bfloat16),
    mesh=_mesh,
    compiler_params=_sc_params,
    scratch_types=[
        pltpu.VMEM_SHARED((NP, D), jnp.bfloat16),
        pltpu.VMEM((NCHUNK, CH), jnp.int32),
        pltpu.VMEM((NCHUNK, CH), jnp.int32),
        pltpu.VMEM((NBUF, CH, D), jnp.bfloat16),
        pltpu.SemaphoreType.DMA,
        pltpu.SemaphoreType.DMA,
    ],
)
def _agg_kernel(y_hbm, row3, col3, zrows_hbm, aggp, agg_sh, row_v, col_v,
                bufs, gsem, ssem):
    c = lax.axis_index("c")
    s = lax.axis_index("s")
    wid = c * NS + s
    pltpu.sync_copy(zrows_hbm, agg_sh.at[pl.ds(s * STRIPE, STRIPE)])
    pltpu.sync_copy(row3.at[wid], row_v)
    pltpu.sync_copy(col3.at[wid], col_v)
    plsc.subcore_barrier()

    def gather_start(j, b):
        pltpu.async_copy(y_hbm.at[row_v.at[j]], bufs.at[b], gsem)

    def gather_wait(b):
        pltpu.make_async_copy(y_hbm.at[row_v.at[0]], bufs.at[b], gsem).wait()

    def scatter_start(j, b):
        pltpu.async_copy(bufs.at[b], agg_sh.at[col_v.at[j]], ssem, add=True)

    def scatter_wait():
        pltpu.make_async_copy(bufs.at[0], agg_sh.at[col_v.at[0]],
                              ssem).wait()

    for j in range(NBUF - 1):
        gather_start(j, j)

    def body(g, carry):
        for b in range(NBUF):
            j = NBUF * g + b
            gather_wait(b)
            scatter_start(j, b)
            scatter_wait()
            gather_start(j + NBUF - 1, (b + NBUF - 1) % NBUF)
        return carry

    NB_STEPS = (NCHUNK - NBUF) // NBUF
    lax.fori_loop(0, NB_STEPS, body, 0)
    done = NB_STEPS * NBUF
    for j in range(done, NCHUNK):
        b = j % NBUF
        gather_wait(b)
        scatter_start(j, b)
        scatter_wait()
        if j + NBUF - 1 < NCHUNK:
            gather_start(j + NBUF - 1, (b + NBUF - 1) % NBUF)
    plsc.subcore_barrier()
    pltpu.sync_copy(agg_sh.at[pl.ds(s * STRIPE, STRIPE)],
                    aggp.at[c].at[pl.ds(s * STRIPE, STRIPE)])


_RB = 1024
_RF = _RB // D


def _discol(degp_ref):
    deg = degp_ref[0] + degp_ref[1] + 1.0
    dis = lax.rsqrt(deg)
    t = jnp.repeat(dis, D, axis=0)
    rr = lax.broadcasted_iota(jnp.int32, (_RB, D), 0)
    cc = lax.broadcasted_iota(jnp.int32, (_RB, D), 1)
    eye = jnp.where(rr % D == cc, 1.0, 0.0)
    return jnp.sum(t * eye, axis=1, keepdims=True)


def _scale_body(x_ref, degp_ref, y_ref):
    y_ref[...] = (x_ref[...] * _discol(degp_ref)).astype(jnp.bfloat16)


def _out_body(aggp_ref, degp_ref, x_ref, w_ref, b_ref, out_ref):
    dis = _discol(degp_ref)
    aggsum = (aggp_ref[0].astype(jnp.float32) +
              aggp_ref[1].astype(jnp.float32))
    m = dis * aggsum + (dis * dis) * x_ref[...]
    out_ref[...] = lax.dot_general(
        m, w_ref[...], (((1,), (0,)), ((), ())),
        preferred_element_type=jnp.float32) + b_ref[...]


_GRID = NP // _RB


def kernel(x, edge_index, W, b):
    row3 = edge_index[0].reshape(NW, NCHUNK, CH)
    col3 = edge_index[1].reshape(NW, NCHUNK, CH)
    ones_f = jnp.ones((CH,), jnp.float32)
    zflat = jnp.zeros((STRIPE,), jnp.float32)
    zrows = jnp.zeros((STRIPE, D), jnp.bfloat16)

    degp = _deg_kernel(row3, ones_f, zflat)

    y = pl.pallas_call(
        _scale_body,
        grid=(_GRID,),
        in_specs=[
            pl.BlockSpec((_RB, D), lambda i: (i, 0)),
            pl.BlockSpec((NC, _RF, D), lambda i: (0, i, 0)),
        ],
        out_specs=pl.BlockSpec((_RB, D), lambda i: (i, 0)),
        out_shape=jax.ShapeDtypeStruct((N, D), jnp.bfloat16),
    )(x, degp)

    aggp = _agg_kernel(y, row3, col3, zrows)

    out = pl.pallas_call(
        _out_body,
        grid=(_GRID,),
        in_specs=[
            pl.BlockSpec((NC, _RB, D), lambda i: (0, i, 0)),
            pl.BlockSpec((NC, _RF, D), lambda i: (0, i, 0)),
            pl.BlockSpec((_RB, D), lambda i: (i, 0)),
            pl.BlockSpec((D, D), lambda i: (0, 0)),
            pl.BlockSpec((1, D), lambda i: (0, 0)),
        ],
        out_specs=pl.BlockSpec((_RB, D), lambda i: (i, 0)),
        out_shape=jax.ShapeDtypeStruct((N, D), jnp.float32),
    )(aggp, degp, x, W, b.reshape(1, D))
    return out
